# baseline (speedup 1.0000x reference)
import functools

import jax
import jax.numpy as jnp
import numpy as np
from jax import lax
from jax.experimental import pallas as pl
from jax.experimental.pallas import tpu as pltpu
from jax.experimental.pallas import tpu_sc as plsc

N = 10000
K = 30
D_PRIM = 20
D_ANG = 6
RBF_K = 16
MAX_D = 20.0
HID = 128
OUT = 20
NPROT = 32
TROW = 32
CHUNK = 128
NB = 1000

_NC = 2
_NS = 16
_NW = _NC * _NS


def _sc_gather(table, idx3d):
    _, ch_per_w, _ = idx3d.shape
    n_rows_total = _NW * ch_per_w
    nj = ch_per_w // 2
    mesh = plsc.VectorSubcoreMesh(core_axis_name="c", subcore_axis_name="s")

    @functools.partial(
        pl.kernel,
        out_type=jax.ShapeDtypeStruct((n_rows_total * CHUNK, TROW), jnp.bfloat16),
        mesh=mesh,
        scratch_types=[
            pltpu.VMEM((ch_per_w, CHUNK), jnp.int32),
            pltpu.VMEM((CHUNK, TROW), jnp.bfloat16),
            pltpu.VMEM((CHUNK, TROW), jnp.bfloat16),
            pltpu.SemaphoreType.DMA,
        ],
        compiler_params=pltpu.CompilerParams(use_tc_tiling_on_sc=False),
    )
    def gather_kernel(table_h, idx_h, out_h, idx_v, buf0, buf1, gsem):
        wid = lax.axis_index("s") * _NC + lax.axis_index("c")
        row0 = wid * (ch_per_w * CHUNK)
        pltpu.sync_copy(idx_h.at[wid], idx_v)
        pltpu.async_copy(table_h.at[idx_v.at[0]], buf0, gsem)

        def step(j, carry):
            pltpu.async_copy(table_h.at[idx_v.at[2 * j + 1]], buf1, gsem)
            pltpu.make_async_copy(table_h.at[idx_v.at[0]], buf0, gsem).wait()
            pltpu.sync_copy(buf0, out_h.at[pl.ds(row0 + (2 * j) * CHUNK, CHUNK)])

            @pl.when(j < nj - 1)
            def _():
                pltpu.async_copy(table_h.at[idx_v.at[2 * j + 2]], buf0, gsem)

            pltpu.make_async_copy(table_h.at[idx_v.at[0]], buf1, gsem).wait()
            pltpu.sync_copy(buf1, out_h.at[pl.ds(row0 + (2 * j + 1) * CHUNK, CHUNK)])
            return carry

        lax.fori_loop(0, nj, step, 0)

    return gather_kernel(table, idx3d)


def _tc_body(g_ref, o_ref, p_ref, pi_ref,
             w1x, w1r, w1x0, w1r0, b1, w2, b2, w3, b3, w4, b4,
             wo1, bo1, wo2, bo2, wo3, bo3, wo4, bo4,
             out_ref, acc_ref):
    i = pl.program_id(0)
    ngrid = pl.num_programs(0)
    bf = jnp.bfloat16

    def dot(a, b):
        return lax.dot_general(a, b, (((1,), (0,)), ((), ())),
                               preferred_element_type=jnp.float32)

    @pl.when(i == 0)
    def _():
        acc_ref[...] = jnp.zeros_like(acc_ref)

    sigma = MAX_D / (RBF_K - 1)
    centers = (lax.broadcasted_iota(jnp.int32, (1, RBF_K), 1)
               .astype(jnp.float32) * np.float32(sigma))
    neg_inv_2s2 = np.float32(-1.0 / (2.0 * sigma * sigma))

    o0 = o_ref[:, 0:1]
    o1 = o_ref[:, 1:2]
    o2 = o_ref[:, 2:3]

    def edge_feats(g):
        x26 = g[:, 0:26]
        d2 = ((o0 - g[:, 26:27].astype(jnp.float32)) ** 2
              + (o1 - g[:, 27:28].astype(jnp.float32)) ** 2
              + (o2 - g[:, 28:29].astype(jnp.float32)) ** 2)
        dist = jnp.sqrt(d2 + 1e-12)
        rbf = jnp.exp((dist - centers) ** 2 * neg_inv_2s2).astype(bf)
        return x26, rbf

    g0 = g_ref[0]
    x26_0, rbf_0 = edge_feats(g0)
    f01 = dot(x26_0, w1x0[...]) + dot(rbf_0, w1r0[...]) + b1[...]

    def edge_mlp(k):
        g = g_ref[k]
        x26, rbf = edge_feats(g)
        hu = jax.nn.relu(dot(x26, w1x[...]) + dot(rbf, w1r[...])
                         + f01).astype(bf)
        hu = jax.nn.relu(dot(hu, w2[...]) + b2[...]).astype(bf)
        hu = jax.nn.relu(dot(hu, w3[...]) + b3[...]).astype(bf)
        pw = dot(hu, w4[...]) + b4[...]
        return (pw[:, 0:HID] * pw[:, HID:HID + 1]).astype(bf)

    def k2step(j, acc):
        prodcat = jnp.concatenate([edge_mlp(2 * j), edge_mlp(2 * j + 1)],
                                  axis=1)
        return acc + dot(prodcat, wo1[j])

    acc = lax.fori_loop(0, K // 2, k2step, jnp.zeros((NB, 300), jnp.float32),
                        unroll=3)

    h = jax.nn.relu(acc + bo1[...])
    h = jax.nn.relu(dot(h, wo2[...]) + bo2[...])
    h = jax.nn.relu(dot(h, wo3[...]) + bo3[...])
    o20 = dot(h, wo4[...]) + bo4[...]

    prim = p_ref[...]
    lane20 = lax.broadcasted_iota(jnp.int32, (NB, OUT), 1)
    mx = jnp.max(prim, axis=1, keepdims=True)
    am = jnp.min(jnp.where(prim == mx, lane20, OUT), axis=1, keepdims=True)
    energy = jnp.sum(jnp.where(lane20 == am, o20, 0.0), axis=1, keepdims=True)

    seg = lax.broadcasted_iota(jnp.int32, (NB, NPROT), 1)
    onehot = (pi_ref[...] == seg).astype(jnp.float32)
    e2 = jnp.concatenate([energy, jnp.ones((NB, 1), jnp.float32)], axis=1)
    contrib = lax.dot_general(onehot, e2, (((0,), (0,)), ((), ())))
    acc_ref[:, 0:2] += contrib

    @pl.when(i == ngrid - 1)
    def _():
        s = acc_ref[:, 0:1]
        c = acc_ref[:, 1:2]
        out_ref[...] = s / jnp.maximum(c, 1.0)


def _tc_main(gt, orientation, primary, prot2d, weights):
    ngrid = N // NB
    const = lambda *shape: pl.BlockSpec(shape, lambda i: tuple(0 for _ in shape))
    in_specs = [
        pl.BlockSpec((K, NB, TROW), lambda i: (0, i, 0)),
        pl.BlockSpec((NB, 3), lambda i: (i, 0)),
        pl.BlockSpec((NB, D_PRIM), lambda i: (i, 0)),
        pl.BlockSpec((NB, 1), lambda i: (i, 0)),
    ] + [const(*w.shape) for w in weights]
    return pl.pallas_call(
        _tc_body,
        grid=(ngrid,),
        in_specs=in_specs,
        out_specs=pl.BlockSpec((NPROT, 1), lambda i: (0, 0)),
        out_shape=jax.ShapeDtypeStruct((NPROT, 1), jnp.float32),
        scratch_shapes=[pltpu.VMEM((NPROT, 128), jnp.float32)],
        compiler_params=pltpu.CompilerParams(
            dimension_semantics=("arbitrary",)),
    )(gt, orientation, primary, prot2d, *weights)


def kernel(primary, gt_ignore, angles, orientation, connections,
           protein_indices, feat_params, weight_params, out_params):
    f32 = jnp.float32
    table = jnp.concatenate(
        [primary, angles, orientation,
         jnp.zeros((N, TROW - D_PRIM - D_ANG - 3), f32)],
        axis=1).astype(jnp.bfloat16)

    idx_t = jnp.transpose(connections).reshape(-1)
    per_round = _NW * CHUNK
    ch_total = -(-(K * N) // per_round) * _NW
    n_pad = ch_total * CHUNK - K * N
    idx_pad = jnp.concatenate(
        [idx_t, jnp.zeros((n_pad,), jnp.int32)]).reshape(
            _NW, ch_total // _NW, CHUNK)

    g_flat = _sc_gather(table, idx_pad)
    gt = g_flat[: K * N].reshape(K, N, TROW)

    (w1f, b1f), (w2f, b2f), (w3f, b3f), (w4f, b4f) = feat_params
    (w1w, b1w), (w2w, b2w), (w3w, b3w), (w4w, b4w) = weight_params
    (wo1, bo1), (wo2, bo2), (wo3, bo3), (wo4, bo4) = out_params
    r = lambda b: b.reshape(1, -1)
    bf = jnp.bfloat16
    c = lambda w: w.astype(bf)
    z128 = jnp.zeros((HID, HID), jnp.float32)
    blk = lambda a, b: jnp.concatenate(
        [jnp.concatenate([a, z128], axis=1),
         jnp.concatenate([z128, b], axis=1)], axis=0)
    w4cat = jnp.concatenate(
        [jnp.concatenate([w4f, jnp.zeros((HID, 1), jnp.float32)], axis=1),
         jnp.concatenate([z128, w4w], axis=1)], axis=0)
    weights = [
        c(jnp.concatenate([w1f[0:26], w1w[0:26]], axis=1)),
        c(jnp.concatenate([w1f[26:42], w1w[26:42]], axis=1)),
        c(jnp.concatenate([w1f[42:68], w1w[42:68]], axis=1)),
        c(jnp.concatenate([w1f[68:84], w1w[68:84]], axis=1)),
        jnp.concatenate([r(b1f), r(b1w)], axis=1),
        c(blk(w2f, w2w)), jnp.concatenate([r(b2f), r(b2w)], axis=1),
        c(blk(w3f, w3w)), jnp.concatenate([r(b3f), r(b3w)], axis=1),
        c(w4cat), jnp.concatenate([r(b4f), r(b4w)], axis=1),
        c(wo1.reshape(K // 2, 2 * HID, 300)), r(bo1), wo2, r(bo2), wo3, r(bo3),
        wo4, r(bo4),
    ]
    prot2d = protein_indices.reshape(N, 1)
    return _tc_main(gt, orientation, primary, prot2d, weights)


# speedup vs baseline: 1.0192x; 1.0192x over previous
import functools

import jax
import jax.numpy as jnp
import numpy as np
from jax import lax
from jax.experimental import pallas as pl
from jax.experimental.pallas import tpu as pltpu
from jax.experimental.pallas import tpu_sc as plsc

N = 10000
K = 30
D_PRIM = 20
D_ANG = 6
RBF_K = 16
MAX_D = 20.0
HID = 128
OUT = 20
NPROT = 32
TROW = 32
CHUNK = 128
NB = 1000

_NC = 2
_NS = 16
_NW = _NC * _NS


def _sc_gather(table, idx3d):
    _, ch_per_w, _ = idx3d.shape
    n_rows_total = _NW * ch_per_w
    nj = ch_per_w // 2
    mesh = plsc.VectorSubcoreMesh(core_axis_name="c", subcore_axis_name="s")

    @functools.partial(
        pl.kernel,
        out_type=jax.ShapeDtypeStruct((n_rows_total * CHUNK, TROW), jnp.bfloat16),
        mesh=mesh,
        scratch_types=[
            pltpu.VMEM((ch_per_w, CHUNK), jnp.int32),
            pltpu.VMEM((CHUNK, TROW), jnp.bfloat16),
            pltpu.VMEM((CHUNK, TROW), jnp.bfloat16),
            pltpu.SemaphoreType.DMA,
        ],
        compiler_params=pltpu.CompilerParams(use_tc_tiling_on_sc=False),
    )
    def gather_kernel(table_h, idx_h, out_h, idx_v, buf0, buf1, gsem):
        wid = lax.axis_index("s") * _NC + lax.axis_index("c")
        row0 = wid * (ch_per_w * CHUNK)
        pltpu.sync_copy(idx_h.at[wid], idx_v)
        pltpu.async_copy(table_h.at[idx_v.at[0]], buf0, gsem)

        def step(j, carry):
            pltpu.async_copy(table_h.at[idx_v.at[2 * j + 1]], buf1, gsem)
            pltpu.make_async_copy(table_h.at[idx_v.at[0]], buf0, gsem).wait()
            pltpu.sync_copy(buf0, out_h.at[pl.ds(row0 + (2 * j) * CHUNK, CHUNK)])

            @pl.when(j < nj - 1)
            def _():
                pltpu.async_copy(table_h.at[idx_v.at[2 * j + 2]], buf0, gsem)

            pltpu.make_async_copy(table_h.at[idx_v.at[0]], buf1, gsem).wait()
            pltpu.sync_copy(buf1, out_h.at[pl.ds(row0 + (2 * j + 1) * CHUNK, CHUNK)])
            return carry

        lax.fori_loop(0, nj, step, 0)

    return gather_kernel(table, idx3d)


def _tc_body(g_ref, o_ref, p_ref, pi_ref,
             w1x, w1r, w1x0, w1r0, b1, w2, b2, w3, b3, w4, b4,
             wo1, bo1, wo2, bo2, wo3, bo3, wo4, bo4,
             out_ref, acc_ref):
    i = pl.program_id(0)
    ngrid = pl.num_programs(0)
    bf = jnp.bfloat16

    def dot(a, b):
        return lax.dot_general(a, b, (((1,), (0,)), ((), ())),
                               preferred_element_type=jnp.float32)

    @pl.when(i == 0)
    def _():
        acc_ref[...] = jnp.zeros_like(acc_ref)

    sigma = MAX_D / (RBF_K - 1)
    centers = (lax.broadcasted_iota(jnp.int32, (1, RBF_K), 1)
               .astype(jnp.float32) * np.float32(sigma))
    neg_inv_2s2 = np.float32(-1.0 / (2.0 * sigma * sigma))

    o0 = o_ref[:, 0:1]
    o1 = o_ref[:, 1:2]
    o2 = o_ref[:, 2:3]

    def edge_feats(g):
        x26 = g[:, 0:26]
        d2 = ((o0 - g[:, 26:27].astype(jnp.float32)) ** 2
              + (o1 - g[:, 27:28].astype(jnp.float32)) ** 2
              + (o2 - g[:, 28:29].astype(jnp.float32)) ** 2)
        dist = jnp.sqrt(d2 + 1e-12)
        rbf = jnp.exp((dist - centers) ** 2 * neg_inv_2s2).astype(bf)
        return x26, rbf

    g0 = g_ref[0]
    x26_0, rbf_0 = edge_feats(g0)
    f01 = dot(x26_0, w1x0[...]) + dot(rbf_0, w1r0[...]) + b1[...]

    def edge_mlp(k):
        g = g_ref[k]
        x26, rbf = edge_feats(g)
        hu = jax.nn.relu(dot(x26, w1x[...]) + dot(rbf, w1r[...])
                         + f01).astype(bf)
        hu = jax.nn.relu(dot(hu, w2[...]) + b2[...]).astype(bf)
        hu = jax.nn.relu(dot(hu, w3[...]) + b3[...]).astype(bf)
        pw = dot(hu, w4[...]) + b4[...]
        return (pw[:, 0:HID] * pw[:, HID:HID + 1]).astype(bf)

    def k2step(j, acc):
        prodcat = jnp.concatenate([edge_mlp(2 * j), edge_mlp(2 * j + 1)],
                                  axis=1)
        return acc + dot(prodcat, wo1[j])

    acc = lax.fori_loop(0, K // 2, k2step, jnp.zeros((NB, 300), jnp.float32),
                        unroll=5)

    h = jax.nn.relu(acc + bo1[...])
    h = jax.nn.relu(dot(h, wo2[...]) + bo2[...])
    h = jax.nn.relu(dot(h, wo3[...]) + bo3[...])
    o20 = dot(h, wo4[...]) + bo4[...]

    prim = p_ref[...]
    lane20 = lax.broadcasted_iota(jnp.int32, (NB, OUT), 1)
    mx = jnp.max(prim, axis=1, keepdims=True)
    am = jnp.min(jnp.where(prim == mx, lane20, OUT), axis=1, keepdims=True)
    energy = jnp.sum(jnp.where(lane20 == am, o20, 0.0), axis=1, keepdims=True)

    seg = lax.broadcasted_iota(jnp.int32, (NB, NPROT), 1)
    onehot = (pi_ref[...] == seg).astype(jnp.float32)
    e2 = jnp.concatenate([energy, jnp.ones((NB, 1), jnp.float32)], axis=1)
    contrib = lax.dot_general(onehot, e2, (((0,), (0,)), ((), ())))
    acc_ref[:, 0:2] += contrib

    @pl.when(i == ngrid - 1)
    def _():
        s = acc_ref[:, 0:1]
        c = acc_ref[:, 1:2]
        out_ref[...] = s / jnp.maximum(c, 1.0)


def _tc_main(gt, orientation, primary, prot2d, weights):
    ngrid = N // NB
    const = lambda *shape: pl.BlockSpec(shape, lambda i: tuple(0 for _ in shape))
    in_specs = [
        pl.BlockSpec((K, NB, TROW), lambda i: (0, i, 0)),
        pl.BlockSpec((NB, 3), lambda i: (i, 0)),
        pl.BlockSpec((NB, D_PRIM), lambda i: (i, 0)),
        pl.BlockSpec((NB, 1), lambda i: (i, 0)),
    ] + [const(*w.shape) for w in weights]
    return pl.pallas_call(
        _tc_body,
        grid=(ngrid,),
        in_specs=in_specs,
        out_specs=pl.BlockSpec((NPROT, 1), lambda i: (0, 0)),
        out_shape=jax.ShapeDtypeStruct((NPROT, 1), jnp.float32),
        scratch_shapes=[pltpu.VMEM((NPROT, 128), jnp.float32)],
        compiler_params=pltpu.CompilerParams(
            dimension_semantics=("arbitrary",)),
    )(gt, orientation, primary, prot2d, *weights)


def kernel(primary, gt_ignore, angles, orientation, connections,
           protein_indices, feat_params, weight_params, out_params):
    f32 = jnp.float32
    table = jnp.concatenate(
        [primary, angles, orientation,
         jnp.zeros((N, TROW - D_PRIM - D_ANG - 3), f32)],
        axis=1).astype(jnp.bfloat16)

    idx_t = jnp.transpose(connections).reshape(-1)
    per_round = _NW * CHUNK
    ch_total = -(-(K * N) // per_round) * _NW
    n_pad = ch_total * CHUNK - K * N
    idx_pad = jnp.concatenate(
        [idx_t, jnp.zeros((n_pad,), jnp.int32)]).reshape(
            _NW, ch_total // _NW, CHUNK)

    g_flat = _sc_gather(table, idx_pad)
    gt = g_flat[: K * N].reshape(K, N, TROW)

    (w1f, b1f), (w2f, b2f), (w3f, b3f), (w4f, b4f) = feat_params
    (w1w, b1w), (w2w, b2w), (w3w, b3w), (w4w, b4w) = weight_params
    (wo1, bo1), (wo2, bo2), (wo3, bo3), (wo4, bo4) = out_params
    r = lambda b: b.reshape(1, -1)
    bf = jnp.bfloat16
    c = lambda w: w.astype(bf)
    z128 = jnp.zeros((HID, HID), jnp.float32)
    blk = lambda a, b: jnp.concatenate(
        [jnp.concatenate([a, z128], axis=1),
         jnp.concatenate([z128, b], axis=1)], axis=0)
    w4cat = jnp.concatenate(
        [jnp.concatenate([w4f, jnp.zeros((HID, 1), jnp.float32)], axis=1),
         jnp.concatenate([z128, w4w], axis=1)], axis=0)
    weights = [
        c(jnp.concatenate([w1f[0:26], w1w[0:26]], axis=1)),
        c(jnp.concatenate([w1f[26:42], w1w[26:42]], axis=1)),
        c(jnp.concatenate([w1f[42:68], w1w[42:68]], axis=1)),
        c(jnp.concatenate([w1f[68:84], w1w[68:84]], axis=1)),
        jnp.concatenate([r(b1f), r(b1w)], axis=1),
        c(blk(w2f, w2w)), jnp.concatenate([r(b2f), r(b2w)], axis=1),
        c(blk(w3f, w3w)), jnp.concatenate([r(b3f), r(b3w)], axis=1),
        c(w4cat), jnp.concatenate([r(b4f), r(b4w)], axis=1),
        c(wo1.reshape(K // 2, 2 * HID, 300)), r(bo1), wo2, r(bo2), wo3, r(bo3),
        wo4, r(bo4),
    ]
    prot2d = protein_indices.reshape(N, 1)
    return _tc_main(gt, orientation, primary, prot2d, weights)
